# Initial kernel scaffold; baseline (speedup 1.0000x reference)
import jax
import jax.numpy as jnp
from jax.experimental import pallas as pl


def kernel(graph_feats, graph_ctrs, graph_pose, sub_ctrs, sub_pose, roi_feat, pre_uv, suc_uv, left_uv, right_uv, params):
    raise NotImplementedError("write your pallas kernel here")


# speedup vs baseline: 15.8191x; 15.8191x over previous
import functools

import jax
import jax.numpy as jnp
from jax import lax
from jax.experimental import pallas as pl
from jax.experimental.pallas import tpu as pltpu
from jax.experimental.pallas import tpu_sc as plsc

N_G = 10000
N_R = 2000
N_MAP = 128
NUM_SCALES = 6
E_SCALE = 10000
E_LR = 2000
N_LAYERS = 4

NC = 2
NS = 16
NW = NC * NS

CAP_T = 2048
CAP = NW * CAP_T
J_AC = CAP_T // 128

C_PER_TILE = 63
G_VECS = N_G // 16
UNROLL = 5

CODE_SHIFT = 14
PAD_C = N_R
PAD_G = N_G
PAD_CODE = (PAD_C << CODE_SHIFT) + PAD_G

ACC_G = 10240
ACC_R = 2048

E_TOT = 2 * NUM_SCALES * E_SCALE + 2 * E_LR
J_B = 31
E_PAD = NW * J_B * 128


def _mesh():
    return plsc.VectorSubcoreMesh(core_axis_name="c", subcore_axis_name="s")


def _make_pair_build():
    @functools.partial(
        pl.kernel,
        out_type=jax.ShapeDtypeStruct((NW * CAP_T,), jnp.int32),
        mesh=_mesh(),
        compiler_params=pltpu.CompilerParams(needs_layout_passes=False),
        scratch_types=[
            pltpu.VMEM((N_R + 16,), jnp.float32),
            pltpu.VMEM((N_R + 16,), jnp.float32),
            pltpu.VMEM((N_G,), jnp.float32),
            pltpu.VMEM((N_G,), jnp.float32),
            pltpu.VMEM((CAP_T + 16,), jnp.int32),
        ],
    )
    def build(sx_h, sy_h, gx_h, gy_h, codes_h, sx, sy, gxv, gyv, codes):
        cid = lax.axis_index("c")
        sid = lax.axis_index("s")
        wid = sid * NC + cid
        pltpu.sync_copy(sx_h, sx)
        pltpu.sync_copy(sy_h, sy)
        pltpu.sync_copy(gx_h, gxv)
        pltpu.sync_copy(gy_h, gyv)

        pad_vec = jnp.full((16,), PAD_CODE, jnp.int32)

        def fill(i, _):
            codes[pl.ds(i * 16, 16)] = pad_vec
            return 0

        lax.fori_loop(0, (CAP_T + 16) // 16, fill, 0)
        iota = lax.iota(jnp.int32, 16)

        def c_body(ci, ptr):
            c = wid * C_PER_TILE + ci

            def run(ptr):
                cx = jnp.full((16,), sx[pl.ds(c, 16)][0], jnp.float32)
                cy = jnp.full((16,), sy[pl.ds(c, 16)][0], jnp.float32)

                def g_body(gi, ptr):
                    base = gi * (16 * UNROLL)
                    ms = []
                    anym = None
                    for u in range(UNROLL):
                        off = base + u * 16
                        dx = gxv[pl.ds(off, 16)] - cx
                        dy = gyv[pl.ds(off, 16)] - cy
                        d2 = dx * dx + dy * dy
                        m = d2 <= 4.0
                        ms.append((off, m))
                        anym = m if anym is None else (anym | m)
                    anyc = plsc.all_reduce_population_count(anym)[0]
                    hit = (anyc > 0) & (ptr <= CAP_T - 16 * UNROLL)

                    def do(p):
                        base_code = c << CODE_SHIFT
                        for off, m in ms:
                            codev = jnp.full((16,), base_code + off,
                                             jnp.int32) + iota
                            plsc.store_compressed(codes.at[pl.ds(p, 16)],
                                                  codev, mask=m)
                            p = p + plsc.all_reduce_population_count(m)[0]
                        return p

                    return lax.cond(hit, do, lambda p: p, ptr)

                return lax.fori_loop(0, G_VECS // UNROLL, g_body, ptr)

            return lax.cond(c < N_R, run, lambda p: p, ptr)

        lax.fori_loop(0, C_PER_TILE, c_body, 0)
        pltpu.sync_copy(codes.at[pl.ds(0, CAP_T)],
                        codes_h.at[pl.ds(wid * CAP_T, CAP_T)])

    return build


def _make_pair_prep():
    @functools.partial(
        pl.kernel,
        out_type=(
            jax.ShapeDtypeStruct((CAP * 4,), jnp.float32),
            jax.ShapeDtypeStruct((NW, J_AC, 128), jnp.int32),
            jax.ShapeDtypeStruct((NW, J_AC, 128), jnp.int32),
        ),
        mesh=_mesh(),
        compiler_params=pltpu.CompilerParams(needs_layout_passes=False),
        scratch_types=[
            pltpu.VMEM((CAP_T,), jnp.int32),
            pltpu.VMEM((4 * ACC_R,), jnp.float32),
            pltpu.VMEM((4 * ACC_G,), jnp.float32),
            pltpu.VMEM((CAP_T * 4,), jnp.float32),
            pltpu.VMEM((J_AC, 128), jnp.int32),
            pltpu.VMEM((J_AC, 128), jnp.int32),
        ],
    )
    def prep(codes_h, spT_h, gpT_h, dpose_h, idxc_h, idxg_h,
             codes, sp, gp, dpv, icv, igv):
        cid = lax.axis_index("c")
        sid = lax.axis_index("s")
        wid = sid * NC + cid
        pltpu.sync_copy(codes_h.at[pl.ds(wid * CAP_T, CAP_T)], codes)
        pltpu.sync_copy(spT_h, sp)
        pltpu.sync_copy(gpT_h, gp)
        iota = lax.iota(jnp.int32, 16)

        def body(i, _):
            code = codes[pl.ds(i * 16, 16)]
            cidx = lax.shift_right_logical(code, CODE_SHIFT)
            gidx = code & ((1 << CODE_SHIFT) - 1)
            icv[i // 8, pl.ds((i % 8) * 16, 16)] = cidx
            igv[i // 8, pl.ds((i % 8) * 16, 16)] = gidx
            li = i * 16 + iota
            for j in range(4):
                spv = plsc.load_gather(sp, [cidx + (j * ACC_R)])
                gpv = plsc.load_gather(gp, [gidx + (j * ACC_G)])
                plsc.store_scatter(dpv, [li * 4 + j], spv - gpv)
            return 0

        lax.fori_loop(0, CAP_T // 16, body, 0)
        pltpu.sync_copy(dpv, dpose_h.at[pl.ds(wid * CAP_T * 4, CAP_T * 4)])
        pltpu.sync_copy(icv, idxc_h.at[wid])
        pltpu.sync_copy(igv, idxg_h.at[wid])

    return prep


def _make_gather(W=N_MAP):
    @functools.partial(
        pl.kernel,
        out_type=jax.ShapeDtypeStruct((CAP, W), jnp.float32),
        mesh=_mesh(),
        compiler_params=pltpu.CompilerParams(needs_layout_passes=False),
        scratch_types=[
            pltpu.VMEM((J_AC, 128), jnp.int32),
            pltpu.VMEM((2, 128, W), jnp.float32),
            pltpu.SemaphoreType.DMA,
            pltpu.SemaphoreType.DMA,
        ],
    )
    def gather(tbl_h, idx_h, out_h, idxv, buf, sem0, sem1):
        cid = lax.axis_index("c")
        sid = lax.axis_index("s")
        wid = sid * NC + cid
        pltpu.sync_copy(idx_h.at[wid], idxv)
        sems = [sem0, sem1]
        descs = [None] * J_AC
        descs[0] = pltpu.async_copy(tbl_h.at[idxv.at[0]], buf.at[0], sems[0])
        for j in range(J_AC):
            if j + 1 < J_AC:
                descs[j + 1] = pltpu.async_copy(
                    tbl_h.at[idxv.at[j + 1]], buf.at[(j + 1) % 2],
                    sems[(j + 1) % 2])
            descs[j].wait()
            pltpu.sync_copy(buf.at[j % 2],
                            out_h.at[pl.ds(wid * CAP_T + j * 128, 128)])

    return gather


def _make_gather_scatter(J, ACC):
    rows_pt = ACC // NS
    nchunk = (rows_pt + 127) // 128

    @functools.partial(
        pl.kernel,
        out_type=jax.ShapeDtypeStruct((NC, ACC, N_MAP), jnp.float32),
        mesh=_mesh(),
        compiler_params=pltpu.CompilerParams(needs_layout_passes=False),
        scratch_types=[
            pltpu.VMEM((J, 128), jnp.int32),
            pltpu.VMEM((J, 128), jnp.int32),
            pltpu.VMEM((2, 128, N_MAP), jnp.float32),
            pltpu.VMEM_SHARED((ACC, N_MAP), jnp.float32),
            pltpu.SemaphoreType.DMA,
            pltpu.SemaphoreType.DMA,
        ],
    )
    def gsca(tbl_h, gidx_h, didx_h, out_h, gidx, didx, buf, acc,
             sem0, sem1):
        cid = lax.axis_index("c")
        sid = lax.axis_index("s")
        wid = sid * NC + cid
        pltpu.sync_copy(gidx_h.at[wid], gidx)
        pltpu.sync_copy(didx_h.at[wid], didx)

        zero16 = jnp.zeros((16,), jnp.float32)
        zbuf = buf.at[0]

        def zf(i, _):
            zbuf[i // 8, pl.ds((i % 8) * 16, 16)] = zero16
            return 0

        lax.fori_loop(0, (128 * N_MAP) // 16, zf, 0)
        base = sid * rows_pt
        off = 0
        for _ in range(nchunk):
            size = min(128, rows_pt - off)
            pltpu.sync_copy(zbuf.at[pl.ds(0, size)],
                            acc.at[pl.ds(base + off, size)])
            off += size
        plsc.subcore_barrier()

        sems = [sem0, sem1]
        descs = [None] * J
        descs[0] = pltpu.async_copy(tbl_h.at[gidx.at[0]], buf.at[0], sems[0])
        for j in range(J):
            if j + 1 < J:
                descs[j + 1] = pltpu.async_copy(
                    tbl_h.at[gidx.at[j + 1]], buf.at[(j + 1) % 2],
                    sems[(j + 1) % 2])
            descs[j].wait()
            pltpu.sync_copy(buf.at[j % 2], acc.at[didx.at[j]], add=True)

        plsc.subcore_barrier()
        off = 0
        for _ in range(nchunk):
            size = min(128, rows_pt - off)
            pltpu.sync_copy(acc.at[pl.ds(base + off, size)],
                            out_h.at[cid].at[pl.ds(base + off, size)])
            off += size

    return gsca


def _gn(x, gamma, beta, eps=1e-5):
    mean = jnp.mean(x, axis=-1, keepdims=True)
    var = jnp.mean((x - mean) ** 2, axis=-1, keepdims=True)
    return (x - mean) / jnp.sqrt(var + eps) * gamma + beta


def _dot(a, b):
    return jnp.dot(a, b, preferred_element_type=jnp.float32)


BLK_P = 2048


def _mlp_body(f_ref, dp_ref, wrel, brel, w0f, w0d, g0, b0, w1, o_ref):
    df = jnp.maximum(_dot(dp_ref[...], wrel[...]) + brel[...], 0.0)
    h = _dot(f_ref[...], w0f[...]) + _dot(df, w0d[...])
    h = jnp.maximum(_gn(h, g0[...], b0[...]), 0.0)
    o_ref[...] = _dot(h, w1[...])


def _pair_mlp(feat, dpose, p, sign):
    row = lambda v: v.reshape(1, N_MAP)
    full = lambda s: pl.BlockSpec(s, lambda i: tuple(0 for _ in s))
    return pl.pallas_call(
        _mlp_body,
        grid=(CAP // BLK_P,),
        in_specs=[
            pl.BlockSpec((BLK_P, N_MAP), lambda i: (i, 0)),
            pl.BlockSpec((BLK_P, 4), lambda i: (i, 0)),
            full((4, N_MAP)),
            full((1, N_MAP)),
            full((N_MAP, N_MAP)),
            full((N_MAP, N_MAP)),
            full((1, N_MAP)),
            full((1, N_MAP)),
            full((N_MAP, N_MAP)),
        ],
        out_specs=pl.BlockSpec((BLK_P, N_MAP), lambda i: (i, 0)),
        out_shape=jax.ShapeDtypeStruct((CAP, N_MAP), jnp.float32),
    )(feat, dpose, p['W_rel'] * sign, row(p['b_rel']),
      p['W_ctx0'][:N_MAP], p['W_ctx0'][N_MAP:],
      row(p['g_ctx0']), row(p['be_ctx0']), p['W_ctx1'])


def _post_A(p0, p1, p):
    row = lambda v: v.reshape(1, N_MAP)
    full = lambda s: pl.BlockSpec(s, lambda i: tuple(0 for _ in s))
    BLK = ACC_G // 8

    def body(p0r, p1r, gn0, bn0, w0, g0, b0, w1, g1, b1, o_ref):
        t = p0r[...] + p1r[...]
        t = jnp.maximum(_gn(t, gn0[...], bn0[...]), 0.0)
        h = jnp.maximum(_gn(_dot(t, w0[...]), g0[...], b0[...]), 0.0)
        h = _gn(_dot(h, w1[...]), g1[...], b1[...])
        o_ref[...] = jnp.maximum(h, 0.0)

    return pl.pallas_call(
        body,
        grid=(8,),
        in_specs=[pl.BlockSpec((BLK, N_MAP), lambda i: (i, 0)),
                  pl.BlockSpec((BLK, N_MAP), lambda i: (i, 0)),
                  full((1, N_MAP)), full((1, N_MAP)),
                  full((N_MAP, N_MAP)), full((1, N_MAP)), full((1, N_MAP)),
                  full((N_MAP, N_MAP)), full((1, N_MAP)), full((1, N_MAP))],
        out_specs=pl.BlockSpec((BLK, N_MAP), lambda i: (i, 0)),
        out_shape=jax.ShapeDtypeStruct((ACC_G, N_MAP), jnp.float32),
    )(p0, p1, row(p['g_norm']), row(p['be_norm']),
      p['W_mlp0'], row(p['g_mlp0']), row(p['be_mlp0']),
      p['W_mlp1'], row(p['g_mlp1']), row(p['be_mlp1']))


def _post_C(p0, p1, roi_pad, p):
    row = lambda v: v.reshape(1, N_MAP)
    full = lambda s: pl.BlockSpec(s, lambda i: tuple(0 for _ in s))
    BLK = ACC_R // 2

    def body(p0r, p1r, roir, wi, gn0, bn0, w0, g0, b0, w1, g1, b1, o_ref):
        t = _dot(roir[...], wi[...]) + p0r[...] + p1r[...]
        t = jnp.maximum(_gn(t, gn0[...], bn0[...]), 0.0)
        h = jnp.maximum(_gn(_dot(t, w0[...]), g0[...], b0[...]), 0.0)
        h = _gn(_dot(h, w1[...]), g1[...], b1[...])
        o_ref[...] = jnp.maximum(h + roir[...], 0.0)

    return pl.pallas_call(
        body,
        grid=(2,),
        in_specs=[pl.BlockSpec((BLK, N_MAP), lambda i: (i, 0)),
                  pl.BlockSpec((BLK, N_MAP), lambda i: (i, 0)),
                  pl.BlockSpec((BLK, N_MAP), lambda i: (i, 0)),
                  full((N_MAP, N_MAP)),
                  full((1, N_MAP)), full((1, N_MAP)),
                  full((N_MAP, N_MAP)), full((1, N_MAP)), full((1, N_MAP)),
                  full((N_MAP, N_MAP)), full((1, N_MAP)), full((1, N_MAP))],
        out_specs=pl.BlockSpec((BLK, N_MAP), lambda i: (i, 0)),
        out_shape=jax.ShapeDtypeStruct((ACC_R, N_MAP), jnp.float32),
    )(p0, p1, roi_pad, p['W_input'], row(p['g_norm']), row(p['be_norm']),
      p['W_mlp0'], row(p['g_mlp0']), row(p['be_mlp0']),
      p['W_mlp1'], row(p['g_mlp1']), row(p['be_mlp1']))


def _mm15(feat, W):
    BLK = ACC_G // 8

    def body(f_ref, w_ref, o_ref):
        o_ref[0] = _dot(f_ref[...], w_ref[0])

    return pl.pallas_call(
        body,
        grid=(8, 15),
        in_specs=[pl.BlockSpec((BLK, N_MAP), lambda j, s: (j, 0)),
                  pl.BlockSpec((1, N_MAP, N_MAP), lambda j, s: (s, 0, 0))],
        out_specs=pl.BlockSpec((1, BLK, N_MAP), lambda j, s: (s, j, 0)),
        out_shape=jax.ShapeDtypeStruct((15, ACC_G, N_MAP), jnp.float32),
    )(feat, W)


def _post_ggn(temp0, p0, p1, res, gn0, bn0, w2, g2, b2):
    row = lambda v: v.reshape(1, N_MAP)
    full = lambda s: pl.BlockSpec(s, lambda i: tuple(0 for _ in s))
    BLK = ACC_G // 8

    def body(t0r, p0r, p1r, resr, gn0r, bn0r, w2r, g2r, b2r, o_ref):
        temp = t0r[...] + p0r[...] + p1r[...]
        t = jnp.maximum(_gn(temp, gn0r[...], bn0r[...]), 0.0)
        t = _gn(_dot(t, w2r[...]), g2r[...], b2r[...])
        o_ref[...] = jnp.maximum(t + resr[...], 0.0)

    return pl.pallas_call(
        body,
        grid=(8,),
        in_specs=[pl.BlockSpec((BLK, N_MAP), lambda i: (i, 0)),
                  pl.BlockSpec((BLK, N_MAP), lambda i: (i, 0)),
                  pl.BlockSpec((BLK, N_MAP), lambda i: (i, 0)),
                  pl.BlockSpec((BLK, N_MAP), lambda i: (i, 0)),
                  full((1, N_MAP)), full((1, N_MAP)),
                  full((N_MAP, N_MAP)), full((1, N_MAP)), full((1, N_MAP))],
        out_specs=pl.BlockSpec((BLK, N_MAP), lambda i: (i, 0)),
        out_shape=jax.ShapeDtypeStruct((ACC_G, N_MAP), jnp.float32),
    )(temp0, p0, p1, res, row(gn0), row(bn0), w2, row(g2), row(b2))


_pair_build = _make_pair_build()
_pair_prep = _make_pair_prep()
_gather_feat = _make_gather(N_MAP)
_gs_A = _make_gather_scatter(J_AC, ACC_G)
_gs_B = _make_gather_scatter(J_B, ACC_G)
_gs_C = _make_gather_scatter(J_AC, ACC_R)


def kernel(graph_feats, graph_ctrs, graph_pose, sub_ctrs, sub_pose, roi_feat,
           pre_uv, suc_uv, left_uv, right_uv, params):
    pA = params['roi2graph']
    pC = params['graph2roi']
    gg = params['ggn']

    subx = jnp.pad(sub_ctrs[:, 0], (0, 16))
    suby = jnp.pad(sub_ctrs[:, 1], (0, 16))
    gx = graph_ctrs[:, 0]
    gy = graph_ctrs[:, 1]
    sposeF = jnp.pad(sub_pose, ((0, ACC_R - N_R), (0, 0))).T.reshape(-1)
    gposeF = jnp.pad(graph_pose, ((0, ACC_G - N_G), (0, 0))).T.reshape(-1)
    roi_pad = jnp.pad(roi_feat, ((0, ACC_R - N_R), (0, 0)))

    codes = _pair_build(subx, suby, gx, gy)
    dposeF, idxC, idxG = _pair_prep(codes, sposeF, gposeF)
    dpose = dposeF.reshape(CAP, 4)
    iota_idx = lax.iota(jnp.int32, CAP).reshape(NW, J_AC, 128)

    featA = _gather_feat(roi_pad, idxC)
    msgsA = _pair_mlp(featA, dpose, pA, 1.0)
    partA = _gs_A(msgsA, iota_idx, idxG)
    feat = _post_A(partA[0], partA[1], pA)

    W_all = jnp.concatenate([
        gg['W_ctr'][:, None],
        jnp.swapaxes(gg['W_pre'], 0, 1),
        jnp.swapaxes(gg['W_suc'], 0, 1),
        gg['W_left'][:, None],
        gg['W_right'][:, None]], axis=1)

    srcs = jnp.concatenate(
        [(1 + k) * ACC_G + pre_uv[k, 1] for k in range(NUM_SCALES)] +
        [(7 + k) * ACC_G + suc_uv[k, 1] for k in range(NUM_SCALES)] +
        [13 * ACC_G + left_uv[1], 14 * ACC_G + right_uv[1]])
    dsts = jnp.concatenate(
        [pre_uv[k, 0] for k in range(NUM_SCALES)] +
        [suc_uv[k, 0] for k in range(NUM_SCALES)] +
        [left_uv[0], right_uv[0]])
    msg_idx = jnp.pad(srcs, (0, E_PAD - E_TOT)).reshape(NW, J_B, 128)
    dst_idx = jnp.pad(dsts, (0, E_PAD - E_TOT),
                      constant_values=N_G).reshape(NW, J_B, 128)

    for i in range(N_LAYERS):
        Y = _mm15(feat, W_all[i])
        Yf = Y.reshape(15 * ACC_G, N_MAP)
        part = _gs_B(Yf, msg_idx, dst_idx)
        feat = _post_ggn(Y[0], part[0], part[1], feat,
                         gg['g_norm'][i], gg['be_norm'][i],
                         gg['W_ctr2'][i], gg['g_ctr2'][i], gg['be_ctr2'][i])

    featC = _gather_feat(feat, idxG)
    msgsC = _pair_mlp(featC, dpose, pC, -1.0)
    partC = _gs_C(msgsC, iota_idx, idxC)
    out = _post_C(partC[0], partC[1], roi_pad, pC)
    return out[:N_R]
